# baseline (speedup 1.0000x reference)
import math

import numpy as np
import jax
import jax.numpy as jnp
from jax.experimental import pallas as pl
from jax.experimental.pallas import tpu as pltpu

_LANE = 128


def _avg_pool_matrix(size, k):
    p = size // k
    M = np.zeros((p, size), np.float32)
    for i in range(p):
        M[i, i * k:(i + 1) * k] = 1.0 / k
    return M


def _bilinear_matrix(out_size, in_size):
    M = np.zeros((out_size, in_size), np.float32)
    if in_size == 1:
        M[:, 0] = 1.0
        return M
    scale = in_size / out_size
    for h in range(out_size):
        src = max((h + 0.5) * scale - 0.5, 0.0)
        i0 = min(int(math.floor(src)), in_size - 1)
        i1 = min(i0 + 1, in_size - 1)
        frac = src - i0
        M[h, i0] += 1.0 - frac
        M[h, i1] += frac
    return M


def _branch_factors(H, W):
    phs, uhs, mwts = [], [], []
    for kh, kw in [(H, W), (H // 2, W // 2), (H // 3, W // 3), (H // 6, W // 6)]:
        Ph, Pw = _avg_pool_matrix(H, kh), _avg_pool_matrix(W, kw)
        Uh, Uw = _bilinear_matrix(H, Ph.shape[0]), _bilinear_matrix(W, Pw.shape[0])
        phs.append(Ph)
        uhs.append(Uh)
        mwts.append((Uw @ Pw).T)
    return phs, uhs, mwts


def _batch_tile(batch, cap=16):
    best = 1
    for bt in range(1, min(batch, cap) + 1):
        if batch % bt == 0 and (batch == 1 or batch // bt >= 2):
            best = bt
    return best


def _spp_body(x_ref, m1_ref, mwt_ref, wid8_ref, uhb_ref, bias_ref, o_ref,
              xc_ref, aa_ref):
    Bt, C, H, W = x_ref.shape
    PHO = m1_ref.shape[0]
    O = PHO // 12
    L = Bt * _LANE

    for b in range(Bt):
        xc_ref[:, :, b * _LANE:b * _LANE + W] = x_ref[b].astype(jnp.bfloat16)
    xcf = xc_ref[...].reshape(C * H, L)

    t3 = jnp.dot(m1_ref[...], xcf,
                 preferred_element_type=jnp.float32).astype(jnp.bfloat16)

    szh = (1, 2, 3, 6)
    off = 0
    for k in range(4):
        lo, hi = off * O, (off + szh[k]) * O
        for b in range(Bt):
            blk = t3[lo:hi, b * _LANE:b * _LANE + W]
            aa_ref[lo:hi, b * _LANE:b * _LANE + W] = jnp.dot(
                blk, mwt_ref[k],
                preferred_element_type=jnp.float32).astype(jnp.bfloat16)
        off += szh[k]

    up = (jnp.dot(uhb_ref[...], aa_ref[...],
                  preferred_element_type=jnp.float32)
          + bias_ref[...]).reshape(O, H, L)

    for hg in range(H // 8):
        xg = xc_ref[:, hg * 8:(hg + 1) * 8, :].reshape(C * 8, L)
        idp = jnp.dot(wid8_ref[...], xg,
                      preferred_element_type=jnp.float32).reshape(O, 8, L)
        tot = idp + up[:, hg * 8:(hg + 1) * 8, :]
        for b in range(Bt):
            o_ref[b, :, hg * 8:(hg + 1) * 8, :] = (
                tot[:, :, b * _LANE:b * _LANE + W])


def kernel(x, weight, bias):
    B, C, H, W = x.shape
    O = weight.shape[0]
    CH, OH = C * H, O * H

    phs, uhs, mwts = _branch_factors(H, W)
    szh = [p.shape[0] for p in phs]
    PH = sum(szh)

    w2d = weight.reshape(O, 5 * C).astype(jnp.float32)

    m1_blocks = []
    for k in range(4):
        wk = w2d[:, (k + 1) * C:(k + 2) * C]
        ph = jnp.asarray(phs[k])
        blk = (ph[:, None, None, :] * wk[None, :, :, None])
        m1_blocks.append(blk.reshape(szh[k] * O, CH))
    m1 = jnp.concatenate(m1_blocks, axis=0)

    uhcat = np.concatenate(uhs, axis=1)
    uhb_np = (np.eye(O, dtype=np.float32)[:, None, None, :]
              * uhcat[None, :, :, None]).reshape(OH, PH * O)

    wid8 = jnp.kron(w2d[:, :C], jnp.eye(8, dtype=jnp.float32))
    bias_col = jnp.repeat(bias.astype(jnp.float32), H).reshape(OH, 1)
    mwt = jnp.asarray(np.stack(mwts), jnp.bfloat16)

    Bt = _batch_tile(B)

    out = pl.pallas_call(
        _spp_body,
        out_shape=jax.ShapeDtypeStruct((B, O, H, W), jnp.float32),
        grid=(B // Bt,),
        in_specs=[
            pl.BlockSpec((Bt, C, H, W), lambda i: (i, 0, 0, 0)),
            pl.BlockSpec((PH * O, CH), lambda i: (0, 0)),
            pl.BlockSpec((4, W, W), lambda i: (0, 0, 0)),
            pl.BlockSpec((O * 8, C * 8), lambda i: (0, 0)),
            pl.BlockSpec((OH, PH * O), lambda i: (0, 0)),
            pl.BlockSpec((OH, 1), lambda i: (0, 0)),
        ],
        out_specs=pl.BlockSpec((Bt, O, H, W), lambda i: (i, 0, 0, 0)),
        scratch_shapes=[
            pltpu.VMEM((C, H, Bt * _LANE), jnp.bfloat16),
            pltpu.VMEM((PH * O, Bt * _LANE), jnp.bfloat16),
        ],
        compiler_params=pltpu.CompilerParams(
            dimension_semantics=("parallel",)),
    )(x.astype(jnp.float32), m1.astype(jnp.bfloat16), mwt,
      wid8.astype(jnp.bfloat16), jnp.asarray(uhb_np, jnp.bfloat16), bias_col)

    return out


# speedup vs baseline: 2.1968x; 2.1968x over previous
import math

import numpy as np
import jax
import jax.numpy as jnp
from jax.experimental import pallas as pl
from jax.experimental.pallas import tpu as pltpu


def _avg_pool_matrix(size, k):
    p = size // k
    M = np.zeros((p, size), np.float32)
    for i in range(p):
        M[i, i * k:(i + 1) * k] = 1.0 / k
    return M


def _bilinear_matrix(out_size, in_size):
    M = np.zeros((out_size, in_size), np.float32)
    if in_size == 1:
        M[:, 0] = 1.0
        return M
    scale = in_size / out_size
    for h in range(out_size):
        src = max((h + 0.5) * scale - 0.5, 0.0)
        i0 = min(int(math.floor(src)), in_size - 1)
        i1 = min(i0 + 1, in_size - 1)
        frac = src - i0
        M[h, i0] += 1.0 - frac
        M[h, i1] += frac
    return M


def _pyramid_operators(H, W):
    p2s, u2ts, sizes = [], [], []
    for kh, kw in [(H, W), (H // 2, W // 2), (H // 3, W // 3), (H // 6, W // 6)]:
        Ph, Pw = _avg_pool_matrix(H, kh), _avg_pool_matrix(W, kw)
        Uh, Uw = _bilinear_matrix(H, Ph.shape[0]), _bilinear_matrix(W, Pw.shape[0])
        p2s.append(np.kron(Ph, Pw))
        u2ts.append(np.kron(Uh, Uw).T)
        sizes.append(p2s[-1].shape[0])
    P = sum(sizes)
    Ppad = 128 * ((P + 127) // 128)
    p2t = np.zeros((H * W, Ppad), np.float32)
    u2t = np.zeros((Ppad, H * W), np.float32)
    masks = np.zeros((4, 1, Ppad), np.float32)
    off = 0
    for k in range(4):
        p2t[:, off:off + sizes[k]] = p2s[k].T
        u2t[off:off + sizes[k], :] = u2ts[k]
        masks[k, 0, off:off + sizes[k]] = 1.0
        off += sizes[k]
    return p2t, u2t, masks


def _batch_tile(batch, cap=32):
    best = 1
    for bt in range(1, min(batch, cap) + 1):
        if batch % bt == 0 and (batch == 1 or batch // bt >= 2):
            best = bt
    return best


def _spp_body(x_ref, p2t_ref, wcat_ref, wid_ref, mask_ref, u2t_ref, bias_ref,
              o_ref):
    Bt, C, HW = x_ref.shape
    O, Ppad = wid_ref.shape[0], p2t_ref.shape[1]

    xf = x_ref[...].reshape(Bt * C, HW)
    g = jnp.dot(xf, p2t_ref[...],
                preferred_element_type=jnp.float32).astype(jnp.bfloat16)

    for b in range(Bt):
        gb = g[b * C:(b + 1) * C, :]
        gstack = jnp.concatenate([gb * mask_ref[k] for k in range(4)], axis=0)
        fb = jnp.dot(wcat_ref[...], gstack,
                     preferred_element_type=jnp.float32)
        o_ref[b] = (jnp.dot(fb.astype(jnp.bfloat16), u2t_ref[...],
                            preferred_element_type=jnp.float32)
                    + jnp.dot(wid_ref[...], x_ref[b],
                              preferred_element_type=jnp.float32)
                    + bias_ref[...])


def kernel(x, weight, bias):
    B, C, H, W = x.shape
    O = weight.shape[0]
    HW = H * W

    p2t_np, u2t_np, masks_np = _pyramid_operators(H, W)
    Ppad = p2t_np.shape[1]

    w2d = weight.reshape(O, 5 * C).astype(jnp.float32)
    wid = w2d[:, :C]
    wcat = w2d[:, C:]
    bias_col = bias.astype(jnp.float32).reshape(O, 1)

    Bt = _batch_tile(B)
    x3 = x.astype(jnp.bfloat16).reshape(B, C, HW)

    out = pl.pallas_call(
        _spp_body,
        out_shape=jax.ShapeDtypeStruct((B, O, HW), jnp.float32),
        grid=(B // Bt,),
        in_specs=[
            pl.BlockSpec((Bt, C, HW), lambda i: (i, 0, 0)),
            pl.BlockSpec((HW, Ppad), lambda i: (0, 0)),
            pl.BlockSpec((O, 4 * C), lambda i: (0, 0)),
            pl.BlockSpec((O, C), lambda i: (0, 0)),
            pl.BlockSpec((4, 1, Ppad), lambda i: (0, 0, 0)),
            pl.BlockSpec((Ppad, HW), lambda i: (0, 0)),
            pl.BlockSpec((O, 1), lambda i: (0, 0)),
        ],
        out_specs=pl.BlockSpec((Bt, O, HW), lambda i: (i, 0, 0)),
        compiler_params=pltpu.CompilerParams(
            dimension_semantics=("parallel",)),
    )(x3, jnp.asarray(p2t_np, jnp.bfloat16), wcat.astype(jnp.bfloat16),
      wid.astype(jnp.bfloat16), jnp.asarray(masks_np, jnp.bfloat16),
      jnp.asarray(u2t_np, jnp.bfloat16), bias_col)

    return out.reshape(B, O, H, W)


# speedup vs baseline: 2.2859x; 1.0405x over previous
import math

import numpy as np
import jax
import jax.numpy as jnp
from jax.experimental import pallas as pl
from jax.experimental.pallas import tpu as pltpu


def _avg_pool_matrix(size, k):
    p = size // k
    M = np.zeros((p, size), np.float32)
    for i in range(p):
        M[i, i * k:(i + 1) * k] = 1.0 / k
    return M


def _bilinear_matrix(out_size, in_size):
    M = np.zeros((out_size, in_size), np.float32)
    if in_size == 1:
        M[:, 0] = 1.0
        return M
    scale = in_size / out_size
    for h in range(out_size):
        src = max((h + 0.5) * scale - 0.5, 0.0)
        i0 = min(int(math.floor(src)), in_size - 1)
        i1 = min(i0 + 1, in_size - 1)
        frac = src - i0
        M[h, i0] += 1.0 - frac
        M[h, i1] += frac
    return M


def _pyramid_operators(H, W):
    p2s, u2ts, sizes = [], [], []
    for kh, kw in [(H, W), (H // 2, W // 2), (H // 3, W // 3), (H // 6, W // 6)]:
        Ph, Pw = _avg_pool_matrix(H, kh), _avg_pool_matrix(W, kw)
        Uh, Uw = _bilinear_matrix(H, Ph.shape[0]), _bilinear_matrix(W, Pw.shape[0])
        p2s.append(np.kron(Ph, Pw))
        u2ts.append(np.kron(Uh, Uw).T)
        sizes.append(p2s[-1].shape[0])
    P = sum(sizes)
    Ppad = 128 * ((P + 127) // 128)
    p2t = np.zeros((H * W, Ppad), np.float32)
    u2t = np.zeros((Ppad, H * W), np.float32)
    masks = np.zeros((4, 1, Ppad), np.float32)
    off = 0
    for k in range(4):
        p2t[:, off:off + sizes[k]] = p2s[k].T
        u2t[off:off + sizes[k], :] = u2ts[k]
        masks[k, 0, off:off + sizes[k]] = 1.0
        off += sizes[k]
    return p2t, u2t, masks


def _batch_tile(batch, cap=32):
    best = 1
    for bt in range(1, min(batch, cap) + 1):
        if batch % bt == 0 and (batch == 1 or batch // bt >= 2):
            best = bt
    return best


def _spp_body(x_ref, p2t_ref, wstack_ref, wid_ref, mask_ref, u2t_ref, bias_ref,
              o_ref):
    Bt, C, HW = x_ref.shape
    O = wid_ref.shape[0]

    xf = x_ref[...].reshape(Bt * C, HW)
    g = jnp.dot(xf, p2t_ref[...], preferred_element_type=jnp.float32)

    for b in range(Bt):
        gb = g[b * C:(b + 1) * C, :]
        rb = jnp.dot(wstack_ref[...], gb, preferred_element_type=jnp.float32)
        fb = rb[0:O, :] * mask_ref[0]
        for k in range(1, 4):
            fb = fb + rb[k * O:(k + 1) * O, :] * mask_ref[k]
        o_ref[b] = (jnp.dot(wid_ref[...], x_ref[b],
                            preferred_element_type=jnp.float32)
                    + bias_ref[...]
                    + jnp.dot(fb, u2t_ref[...],
                              preferred_element_type=jnp.float32))


def kernel(x, weight, bias):
    B, C, H, W = x.shape
    O = weight.shape[0]
    HW = H * W

    p2t_np, u2t_np, masks_np = _pyramid_operators(H, W)
    Ppad = p2t_np.shape[1]

    w2d = weight.reshape(O, 5 * C).astype(jnp.float32)
    wid = w2d[:, :C]
    wstack = jnp.concatenate([w2d[:, (k + 1) * C:(k + 2) * C]
                              for k in range(4)], axis=0)
    bias_col = bias.astype(jnp.float32).reshape(O, 1)

    Bt = _batch_tile(B)
    x3 = x.astype(jnp.float32).reshape(B, C, HW)

    out = pl.pallas_call(
        _spp_body,
        out_shape=jax.ShapeDtypeStruct((B, O, HW), jnp.float32),
        grid=(B // Bt,),
        in_specs=[
            pl.BlockSpec((Bt, C, HW), lambda i: (i, 0, 0)),
            pl.BlockSpec((HW, Ppad), lambda i: (0, 0)),
            pl.BlockSpec((4 * O, C), lambda i: (0, 0)),
            pl.BlockSpec((O, C), lambda i: (0, 0)),
            pl.BlockSpec((4, 1, Ppad), lambda i: (0, 0, 0)),
            pl.BlockSpec((Ppad, HW), lambda i: (0, 0)),
            pl.BlockSpec((O, 1), lambda i: (0, 0)),
        ],
        out_specs=pl.BlockSpec((Bt, O, HW), lambda i: (i, 0, 0)),
        compiler_params=pltpu.CompilerParams(
            dimension_semantics=("parallel",)),
    )(x3, jnp.asarray(p2t_np), wstack, wid, jnp.asarray(masks_np),
      jnp.asarray(u2t_np), bias_col)

    return out.reshape(B, O, H, W)


# speedup vs baseline: 2.6288x; 1.1500x over previous
import math

import numpy as np
import jax
import jax.numpy as jnp
from jax.experimental import pallas as pl
from jax.experimental.pallas import tpu as pltpu


def _avg_pool_matrix(size, k):
    p = size // k
    M = np.zeros((p, size), np.float32)
    for i in range(p):
        M[i, i * k:(i + 1) * k] = 1.0 / k
    return M


def _bilinear_matrix(out_size, in_size):
    M = np.zeros((out_size, in_size), np.float32)
    if in_size == 1:
        M[:, 0] = 1.0
        return M
    scale = in_size / out_size
    for h in range(out_size):
        src = max((h + 0.5) * scale - 0.5, 0.0)
        i0 = min(int(math.floor(src)), in_size - 1)
        i1 = min(i0 + 1, in_size - 1)
        frac = src - i0
        M[h, i0] += 1.0 - frac
        M[h, i1] += frac
    return M


def _pyramid_operators(H, W):
    p2s, u2ts, sizes = [], [], []
    for kh, kw in [(H, W), (H // 2, W // 2), (H // 3, W // 3), (H // 6, W // 6)]:
        Ph, Pw = _avg_pool_matrix(H, kh), _avg_pool_matrix(W, kw)
        Uh, Uw = _bilinear_matrix(H, Ph.shape[0]), _bilinear_matrix(W, Pw.shape[0])
        p2s.append(np.kron(Ph, Pw))
        u2ts.append(np.kron(Uh, Uw).T)
        sizes.append(p2s[-1].shape[0])
    P = sum(sizes)
    Ppad = 128 * ((P + 127) // 128)
    p2t = np.zeros((H * W, Ppad), np.float32)
    u2t = np.zeros((Ppad, H * W), np.float32)
    masks = np.zeros((4, 1, Ppad), np.float32)
    off = 0
    for k in range(4):
        p2t[:, off:off + sizes[k]] = p2s[k].T
        u2t[off:off + sizes[k], :] = u2ts[k]
        masks[k, 0, off:off + sizes[k]] = 1.0
        off += sizes[k]
    return p2t, u2t, masks


def _batch_tile(batch, cap=128):
    best = 1
    for bt in range(1, min(batch, cap) + 1):
        if batch % bt == 0 and (batch == 1 or batch // bt >= 2):
            best = bt
    return best


def _spp_body(x_ref, p2t_ref, wstack_ref, wid_ref, mask_ref, u2t_ref, bias_ref,
              o_ref):
    Bt, C, HW = x_ref.shape
    O = wid_ref.shape[0]

    xf = x_ref[...].reshape(Bt * C, HW)
    g = jnp.dot(xf, p2t_ref[...], preferred_element_type=jnp.float32)

    for b in range(Bt):
        gb = g[b * C:(b + 1) * C, :]
        rb = jnp.dot(wstack_ref[...], gb, preferred_element_type=jnp.float32)
        fb = rb[0:O, :] * mask_ref[0]
        for k in range(1, 4):
            fb = fb + rb[k * O:(k + 1) * O, :] * mask_ref[k]
        o_ref[b] = (jnp.dot(wid_ref[...], x_ref[b],
                            preferred_element_type=jnp.float32)
                    + bias_ref[...]
                    + jnp.dot(fb, u2t_ref[...],
                              preferred_element_type=jnp.float32))


def kernel(x, weight, bias):
    B, C, H, W = x.shape
    O = weight.shape[0]
    HW = H * W

    p2t_np, u2t_np, masks_np = _pyramid_operators(H, W)
    Ppad = p2t_np.shape[1]

    w2d = weight.reshape(O, 5 * C).astype(jnp.float32)
    wid = w2d[:, :C]
    wstack = jnp.concatenate([w2d[:, (k + 1) * C:(k + 2) * C]
                              for k in range(4)], axis=0)
    bias_col = bias.astype(jnp.float32).reshape(O, 1)

    Bt = _batch_tile(B)
    x3 = x.astype(jnp.float32).reshape(B, C, HW)

    out = pl.pallas_call(
        _spp_body,
        out_shape=jax.ShapeDtypeStruct((B, O, HW), jnp.float32),
        grid=(B // Bt,),
        in_specs=[
            pl.BlockSpec((Bt, C, HW), lambda i: (i, 0, 0)),
            pl.BlockSpec((HW, Ppad), lambda i: (0, 0)),
            pl.BlockSpec((4 * O, C), lambda i: (0, 0)),
            pl.BlockSpec((O, C), lambda i: (0, 0)),
            pl.BlockSpec((4, 1, Ppad), lambda i: (0, 0, 0)),
            pl.BlockSpec((Ppad, HW), lambda i: (0, 0)),
            pl.BlockSpec((O, 1), lambda i: (0, 0)),
        ],
        out_specs=pl.BlockSpec((Bt, O, HW), lambda i: (i, 0, 0)),
        compiler_params=pltpu.CompilerParams(
            dimension_semantics=("parallel",)),
    )(x3, jnp.asarray(p2t_np), wstack, wid, jnp.asarray(masks_np),
      jnp.asarray(u2t_np), bias_col)

    return out.reshape(B, O, H, W)


# speedup vs baseline: 3.1154x; 1.1851x over previous
import math

import numpy as np
import jax
import jax.numpy as jnp
from jax.experimental import pallas as pl
from jax.experimental.pallas import tpu as pltpu


def _avg_pool_matrix(size, k):
    p = size // k
    M = np.zeros((p, size), np.float32)
    for i in range(p):
        M[i, i * k:(i + 1) * k] = 1.0 / k
    return M


def _bilinear_matrix(out_size, in_size):
    M = np.zeros((out_size, in_size), np.float32)
    if in_size == 1:
        M[:, 0] = 1.0
        return M
    scale = in_size / out_size
    for h in range(out_size):
        src = max((h + 0.5) * scale - 0.5, 0.0)
        i0 = min(int(math.floor(src)), in_size - 1)
        i1 = min(i0 + 1, in_size - 1)
        frac = src - i0
        M[h, i0] += 1.0 - frac
        M[h, i1] += frac
    return M


def _pyramid_operators(H, W):
    p2s, u2ts, sizes = [], [], []
    for kh, kw in [(H, W), (H // 2, W // 2), (H // 3, W // 3), (H // 6, W // 6)]:
        Ph, Pw = _avg_pool_matrix(H, kh), _avg_pool_matrix(W, kw)
        Uh, Uw = _bilinear_matrix(H, Ph.shape[0]), _bilinear_matrix(W, Pw.shape[0])
        p2s.append(np.kron(Ph, Pw))
        u2ts.append(np.kron(Uh, Uw).T)
        sizes.append(p2s[-1].shape[0])
    P = sum(sizes)
    Ppad = 128 * ((P + 127) // 128)
    p2t = np.zeros((H * W, Ppad), np.float32)
    u2t = np.zeros((Ppad, H * W), np.float32)
    masks = np.zeros((4, 1, Ppad), np.float32)
    off = 0
    for k in range(4):
        p2t[:, off:off + sizes[k]] = p2s[k].T
        u2t[off:off + sizes[k], :] = u2ts[k]
        masks[k, 0, off:off + sizes[k]] = 1.0
        off += sizes[k]
    return p2t, u2t, masks


def _batch_tile(batch, cap=128):
    best = 1
    for bt in range(1, min(batch, cap) + 1):
        if batch % bt == 0 and (batch == 1 or batch // bt >= 2):
            best = bt
    return best


def _spp_body(x_ref, p2t_ref, wstack_ref, wid8_ref, mask_ref, u2t_ref,
              bias_ref, o_ref, f_ref):
    Bt, C, HW = x_ref.shape
    O = wstack_ref.shape[0] // 4

    xf = x_ref[...].reshape(Bt * C, HW)
    g = jnp.dot(xf, p2t_ref[...], preferred_element_type=jnp.float32)

    for b in range(Bt):
        gb = g[b * C:(b + 1) * C, :]
        rb = jnp.dot(wstack_ref[...], gb, preferred_element_type=jnp.float32)
        fb = rb[0:O, :] * mask_ref[0]
        for k in range(1, 4):
            fb = fb + rb[k * O:(k + 1) * O, :] * mask_ref[k]
        f_ref[b * O:(b + 1) * O, :] = fb

    up3 = jnp.dot(f_ref[...], u2t_ref[...],
                  preferred_element_type=jnp.float32).reshape(Bt, O, HW)

    for i in range(Bt // 8):
        xg = xf[i * 8 * C:(i + 1) * 8 * C, :]
        idp = jnp.dot(wid8_ref[...], xg,
                      preferred_element_type=jnp.float32).reshape(8, O, HW)
        o_ref[i * 8:(i + 1) * 8] = (idp + up3[i * 8:(i + 1) * 8]
                                    + bias_ref[...])


def kernel(x, weight, bias):
    B, C, H, W = x.shape
    O = weight.shape[0]
    HW = H * W

    p2t_np, u2t_np, masks_np = _pyramid_operators(H, W)
    Ppad = p2t_np.shape[1]

    w2d = weight.reshape(O, 5 * C).astype(jnp.float32)
    wid8 = jnp.kron(jnp.eye(8, dtype=jnp.float32), w2d[:, :C])
    wstack = jnp.concatenate([w2d[:, (k + 1) * C:(k + 2) * C]
                              for k in range(4)], axis=0)
    bias_col = bias.astype(jnp.float32).reshape(1, O, 1)

    Bt = _batch_tile(B)
    x3 = x.astype(jnp.float32).reshape(B, C, HW)

    out = pl.pallas_call(
        _spp_body,
        out_shape=jax.ShapeDtypeStruct((B, O, HW), jnp.float32),
        grid=(B // Bt,),
        in_specs=[
            pl.BlockSpec((Bt, C, HW), lambda i: (i, 0, 0)),
            pl.BlockSpec((HW, Ppad), lambda i: (0, 0)),
            pl.BlockSpec((4 * O, C), lambda i: (0, 0)),
            pl.BlockSpec((8 * O, 8 * C), lambda i: (0, 0)),
            pl.BlockSpec((4, 1, Ppad), lambda i: (0, 0, 0)),
            pl.BlockSpec((Ppad, HW), lambda i: (0, 0)),
            pl.BlockSpec((1, O, 1), lambda i: (0, 0, 0)),
        ],
        out_specs=pl.BlockSpec((Bt, O, HW), lambda i: (i, 0, 0)),
        scratch_shapes=[
            pltpu.VMEM((Bt * O, Ppad), jnp.float32),
        ],
        compiler_params=pltpu.CompilerParams(
            dimension_semantics=("parallel",)),
    )(x3, jnp.asarray(p2t_np), wstack, wid8, jnp.asarray(masks_np),
      jnp.asarray(u2t_np), bias_col)

    return out.reshape(B, O, H, W)
